# Initial kernel scaffold; baseline (speedup 1.0000x reference)
import jax
import jax.numpy as jnp
from jax.experimental import pallas as pl


def kernel(x, h, g, edge_idx, edge_type, W1, b1, weights_att):
    raise NotImplementedError("write your pallas kernel here")


# speedup vs baseline: 3.0778x; 3.0778x over previous
import jax
import jax.numpy as jnp
from jax import lax
from jax.experimental import pallas as pl
from jax.experimental.pallas import tpu as pltpu
from jax.experimental.pallas import tpu_sc as plsc


def _lock_acquire(lock_ref):
    ticket = plsc.fetch_and_add(lock_ref.at[0], 1, subcore_id=0)
    serving = plsc.fetch_and_add(lock_ref.at[1], 0, subcore_id=0)

    def spin(cur):
        return plsc.fetch_and_add(lock_ref.at[1], 0, subcore_id=0)

    lax.while_loop(lambda cur: cur != ticket, spin, serving)


def _lock_release(lock_ref):
    plsc.fetch_and_add(lock_ref.at[1], 1, subcore_id=0)


_LOG2E = 1.4426950408889634
_LN2 = 0.6931471805599453


def _exp_sc(x):
    t = x * _LOG2E
    t = jnp.minimum(jnp.maximum(t, -126.0), 126.0)
    u = t + 512.5
    ni = u.astype(jnp.int32)
    nf = ni.astype(jnp.float32) - 512.0
    y = (t - nf) * _LN2
    p = 1.0 + y * (1.0 + y * (0.5 + y * (1.0 / 6.0 + y * (
        1.0 / 24.0 + y * (1.0 / 120.0 + y * (1.0 / 720.0))))))
    bits = (ni + (127 - 512)) << 23
    s = plsc.bitcast(bits, jnp.float32)
    return p * s


N = 10000
E = 320000
R = 100
HEADS = 2
OUT = 32
D = HEADS * OUT
IN_H = 128
IN_G = 16
NEG = 0.01

NC = 2
NS = 16
NW = NC * NS
TPE = E // NW
C = 80
NCHUNK = TPE // C
NP = 10240
NPT = NP // NS

NB = 1000
NGRID = N // NB


def _k1_body(h_ref, g_ref, wa_ref, wb_ref, wc_ref, b1_ref, watt_ref,
             u_ref, v_ref, at_ref, gt_ref, ag_ref):
    ub = jnp.dot(h_ref[...], wa_ref[...], preferred_element_type=jnp.float32)
    vb = jnp.dot(h_ref[...], wb_ref[...], preferred_element_type=jnp.float32)
    u_ref[...] = ub
    v_ref[...] = vb
    w0 = watt_ref[0:1, :]
    w1 = watt_ref[1:2, :]
    au0 = jnp.sum(ub[:, :OUT] * w0, axis=1, keepdims=True)
    au1 = jnp.sum(ub[:, OUT:] * w1, axis=1, keepdims=True)
    av0 = jnp.sum(vb[:, :OUT] * w0, axis=1, keepdims=True)
    av1 = jnp.sum(vb[:, OUT:] * w1, axis=1, keepdims=True)
    at_ref[...] = jnp.concatenate([au0, au1, av0, av1], axis=1)

    @pl.when(pl.program_id(0) == 0)
    def _():
        gt = jnp.dot(g_ref[...], wc_ref[...],
                     preferred_element_type=jnp.float32) + b1_ref[...]
        gt_ref[...] = gt
        ag0 = jnp.sum(gt[:, :OUT] * w0, axis=1, keepdims=True)
        ag1 = jnp.sum(gt[:, OUT:] * w1, axis=1, keepdims=True)
        ag_ref[...] = jnp.concatenate([ag0, ag1], axis=1)


_k1 = pl.pallas_call(
    _k1_body,
    grid=(NGRID,),
    in_specs=[
        pl.BlockSpec((NB, IN_H), lambda i: (i, 0)),
        pl.BlockSpec((R, IN_G), lambda i: (0, 0)),
        pl.BlockSpec((IN_H, D), lambda i: (0, 0)),
        pl.BlockSpec((IN_H, D), lambda i: (0, 0)),
        pl.BlockSpec((IN_G, D), lambda i: (0, 0)),
        pl.BlockSpec((1, D), lambda i: (0, 0)),
        pl.BlockSpec((HEADS, OUT), lambda i: (0, 0)),
    ],
    out_specs=[
        pl.BlockSpec((NB, D), lambda i: (i, 0)),
        pl.BlockSpec((NB, D), lambda i: (i, 0)),
        pl.BlockSpec((NB, 4), lambda i: (i, 0)),
        pl.BlockSpec((R, D), lambda i: (0, 0)),
        pl.BlockSpec((R, HEADS), lambda i: (0, 0)),
    ],
    out_shape=[
        jax.ShapeDtypeStruct((N, D), jnp.float32),
        jax.ShapeDtypeStruct((N, D), jnp.float32),
        jax.ShapeDtypeStruct((N, 4), jnp.float32),
        jax.ShapeDtypeStruct((R, D), jnp.float32),
        jax.ShapeDtypeStruct((R, HEADS), jnp.float32),
    ],
)


_sc_mesh = plsc.VectorSubcoreMesh(core_axis_name="c", subcore_axis_name="s")


def _k2_body(row_hbm, col_hbm, et_hbm, at_hbm, ag_hbm, z1_hbm,
             ev0_hbm, ev1_hbm, spart_hbm,
             at_v, ag_v, rows_v, cols_v, et_v, ev0_v, ev1_v, lock_ref,
             s0_sh, s1_sh):
    cid = lax.axis_index("c")
    sid = lax.axis_index("s")
    wid = sid * NC + cid

    @pl.when(sid == 0)
    def _():
        lock_ref[0] = 0
        lock_ref[1] = 0

    pltpu.sync_copy(at_hbm, at_v)
    pltpu.sync_copy(ag_hbm, ag_v)
    pltpu.sync_copy(z1_hbm.at[pl.ds(sid * NPT, NPT)],
                    s0_sh.at[pl.ds(sid * NPT, NPT)])
    pltpu.sync_copy(z1_hbm.at[pl.ds(sid * NPT, NPT)],
                    s1_sh.at[pl.ds(sid * NPT, NPT)])
    plsc.subcore_barrier()

    def chunk(k, carry):
        base = wid * TPE + k * C
        pltpu.sync_copy(row_hbm.at[pl.ds(base, C)], rows_v)
        pltpu.sync_copy(col_hbm.at[pl.ds(base, C)], cols_v)
        pltpu.sync_copy(et_hbm.at[pl.ds(base, C)], et_v)
        for g in range(C // 16):
            sl = pl.ds(g * 16, 16)
            rows4 = rows_v[sl] * 4
            cols4 = cols_v[sl] * 4
            ets2 = et_v[sl] * 2
            a0 = (plsc.load_gather(at_v, [rows4])
                  + plsc.load_gather(at_v, [cols4 + 2])
                  + plsc.load_gather(ag_v, [ets2]))
            a1 = (plsc.load_gather(at_v, [rows4 + 1])
                  + plsc.load_gather(at_v, [cols4 + 3])
                  + plsc.load_gather(ag_v, [ets2 + 1]))
            a0 = jnp.maximum(a0, a0 * NEG)
            a1 = jnp.maximum(a1, a1 * NEG)
            ev0_v[sl] = _exp_sc(a0)
            ev1_v[sl] = _exp_sc(a1)
        pltpu.sync_copy(ev0_v, ev0_hbm.at[pl.ds(base, C)])
        pltpu.sync_copy(ev1_v, ev1_hbm.at[pl.ds(base, C)])
        _lock_acquire(lock_ref)
        pltpu.sync_copy(ev0_v, s0_sh.at[rows_v], add=True)
        pltpu.sync_copy(ev1_v, s1_sh.at[rows_v], add=True)
        _lock_release(lock_ref)
        return carry

    lax.fori_loop(0, NCHUNK, chunk, 0)
    plsc.subcore_barrier()
    pltpu.sync_copy(s0_sh.at[pl.ds(sid * NPT, NPT)],
                    spart_hbm.at[pl.ds((cid * HEADS + 0) * NP + sid * NPT,
                                       NPT)])
    pltpu.sync_copy(s1_sh.at[pl.ds(sid * NPT, NPT)],
                    spart_hbm.at[pl.ds((cid * HEADS + 1) * NP + sid * NPT,
                                       NPT)])


_k2 = pl.kernel(
    _k2_body,
    out_type=[
        jax.ShapeDtypeStruct((E,), jnp.float32),
        jax.ShapeDtypeStruct((E,), jnp.float32),
        jax.ShapeDtypeStruct((NC * HEADS * NP,), jnp.float32),
    ],
    mesh=_sc_mesh,
    compiler_params=pltpu.CompilerParams(
        needs_layout_passes=False, use_tc_tiling_on_sc=False),
    scratch_types=[
        pltpu.VMEM((N * 4,), jnp.float32),
        pltpu.VMEM((R * HEADS,), jnp.float32),
        pltpu.VMEM((C,), jnp.int32),
        pltpu.VMEM((C,), jnp.int32),
        pltpu.VMEM((C,), jnp.int32),
        pltpu.VMEM((C,), jnp.float32),
        pltpu.VMEM((C,), jnp.float32),
        pltpu.SMEM((2,), jnp.int32),
        pltpu.VMEM_SHARED((NP,), jnp.float32),
        pltpu.VMEM_SHARED((NP,), jnp.float32),
    ],
)


def _k4_body(row_hbm, col_hbm, et_hbm, ev0_hbm, ev1_hbm, spart_hbm,
             u_hbm, gt_hbm, z1_hbm, z2_hbm,
             accp_hbm, sap_hbm,
             s0_v, s1_v, tmp_v, rows_v, cols_v, et_v, ev0_v, ev1_v,
             al0_v, al1_v, u_v, gt_v, ctr_v, idx1_v, sem1, sem2, lock_ref,
             acc_sh, sa0_sh, sa1_sh):
    cid = lax.axis_index("c")
    sid = lax.axis_index("s")
    wid = sid * NC + cid

    @pl.when(sid == 0)
    def _():
        lock_ref[0] = 0
        lock_ref[1] = 0

    pltpu.sync_copy(spart_hbm.at[pl.ds(0 * NP, NP)], s0_v)
    pltpu.sync_copy(spart_hbm.at[pl.ds(2 * NP, NP)], tmp_v)

    def addl(i, carry):
        sl = pl.ds(i * 16, 16)
        s0_v[sl] = s0_v[sl] + tmp_v[sl]
        return carry

    lax.fori_loop(0, NP // 16, addl, 0)
    pltpu.sync_copy(spart_hbm.at[pl.ds(1 * NP, NP)], s1_v)
    pltpu.sync_copy(spart_hbm.at[pl.ds(3 * NP, NP)], tmp_v)

    def addl2(i, carry):
        sl = pl.ds(i * 16, 16)
        s1_v[sl] = s1_v[sl] + tmp_v[sl]
        return carry

    lax.fori_loop(0, NP // 16, addl2, 0)

    pltpu.sync_copy(z2_hbm.at[pl.ds(sid * NPT, NPT)],
                    acc_sh.at[pl.ds(sid * NPT, NPT)])
    pltpu.sync_copy(z1_hbm.at[pl.ds(sid * NPT, NPT)],
                    sa0_sh.at[pl.ds(sid * NPT, NPT)])
    pltpu.sync_copy(z1_hbm.at[pl.ds(sid * NPT, NPT)],
                    sa1_sh.at[pl.ds(sid * NPT, NPT)])
    plsc.subcore_barrier()

    def chunk(k, carry):
        base = wid * TPE + k * C
        pltpu.sync_copy(row_hbm.at[pl.ds(base, C)], rows_v)
        pltpu.sync_copy(col_hbm.at[pl.ds(base, C)], cols_v)
        pltpu.sync_copy(et_hbm.at[pl.ds(base, C)], et_v)
        pltpu.sync_copy(ev0_hbm.at[pl.ds(base, C)], ev0_v)
        pltpu.sync_copy(ev1_hbm.at[pl.ds(base, C)], ev1_v)
        cp1 = pltpu.async_copy(u_hbm.at[rows_v], u_v, sem1)
        cp2 = pltpu.async_copy(gt_hbm.at[et_v], gt_v, sem2)
        cp1.wait()
        cp2.wait()
        for g in range(C // 16):
            sl = pl.ds(g * 16, 16)
            rows = rows_v[sl]
            s0 = plsc.load_gather(s0_v, [rows])
            s1 = plsc.load_gather(s1_v, [rows])
            al0_v[sl] = ev0_v[sl] / s0
            al1_v[sl] = ev1_v[sl] / s1
        for e in range(C):
            efull = jnp.full((16,), e, jnp.int32)
            b0 = plsc.load_gather(al0_v, [efull])
            b1v = plsc.load_gather(al1_v, [efull])
            for j in range(D // 16):
                sl = pl.ds(j * 16, 16)
                uv = u_v[e, sl]
                tv = gt_v[e, sl]
                ctr_v[e, sl] = (uv + tv) * (b0 if j < 2 else b1v)
        _lock_acquire(lock_ref)

        def rowcopy(e, carry2):
            cb = plsc.load_gather(cols_v, [jnp.full((16,), e, jnp.int32)])
            plsc.store_scatter(idx1_v, [jnp.zeros((16,), jnp.int32)], cb)
            pltpu.sync_copy(ctr_v.at[pl.ds(e, 1)], acc_sh.at[idx1_v],
                            add=True)
            return carry2

        lax.fori_loop(0, C, rowcopy, 0)
        pltpu.sync_copy(al0_v, sa0_sh.at[cols_v], add=True)
        pltpu.sync_copy(al1_v, sa1_sh.at[cols_v], add=True)
        _lock_release(lock_ref)
        return carry

    lax.fori_loop(0, NCHUNK, chunk, 0)
    plsc.subcore_barrier()
    pltpu.sync_copy(acc_sh.at[pl.ds(sid * NPT, NPT)], accp_hbm.at[cid, sid])
    pltpu.sync_copy(sa0_sh.at[pl.ds(sid * NPT, NPT)],
                    sap_hbm.at[pl.ds((cid * HEADS + 0) * NP + sid * NPT,
                                     NPT)])
    pltpu.sync_copy(sa1_sh.at[pl.ds(sid * NPT, NPT)],
                    sap_hbm.at[pl.ds((cid * HEADS + 1) * NP + sid * NPT,
                                     NPT)])


_k4 = pl.kernel(
    _k4_body,
    out_type=[
        jax.ShapeDtypeStruct((NC, NS, NPT, D), jnp.float32),
        jax.ShapeDtypeStruct((NC * HEADS * NP,), jnp.float32),
    ],
    mesh=_sc_mesh,
    compiler_params=pltpu.CompilerParams(
        needs_layout_passes=False, use_tc_tiling_on_sc=False),
    scratch_types=[
        pltpu.VMEM((NP,), jnp.float32),
        pltpu.VMEM((NP,), jnp.float32),
        pltpu.VMEM((NP,), jnp.float32),
        pltpu.VMEM((C,), jnp.int32),
        pltpu.VMEM((C,), jnp.int32),
        pltpu.VMEM((C,), jnp.int32),
        pltpu.VMEM((C,), jnp.float32),
        pltpu.VMEM((C,), jnp.float32),
        pltpu.VMEM((C,), jnp.float32),
        pltpu.VMEM((C,), jnp.float32),
        pltpu.VMEM((C, D), jnp.float32),
        pltpu.VMEM((C, D), jnp.float32),
        pltpu.VMEM((C, D), jnp.float32),
        pltpu.VMEM((1,), jnp.int32),
        pltpu.SemaphoreType.DMA,
        pltpu.SemaphoreType.DMA,
        pltpu.SMEM((2,), jnp.int32),
        pltpu.VMEM_SHARED((NP, D), jnp.float32),
        pltpu.VMEM_SHARED((NP,), jnp.float32),
        pltpu.VMEM_SHARED((NP,), jnp.float32),
    ],
)


def _k5_body(a0_ref, a1_ref, sat_ref, v_ref, out_ref):
    acc = a0_ref[...] + a1_ref[...]
    sat = sat_ref[...]
    sa0 = sat[:, 0:1] + sat[:, 2:3]
    sa1 = sat[:, 1:2] + sat[:, 3:4]
    vv = v_ref[...]
    m = jnp.concatenate([sa0 * vv[:, :OUT], sa1 * vv[:, OUT:]], axis=1)
    out_ref[...] = acc + m


_k5 = pl.pallas_call(
    _k5_body,
    grid=(NGRID,),
    in_specs=[
        pl.BlockSpec((NB, D), lambda i: (i, 0)),
        pl.BlockSpec((NB, D), lambda i: (i, 0)),
        pl.BlockSpec((NB, 4), lambda i: (i, 0)),
        pl.BlockSpec((NB, D), lambda i: (i, 0)),
    ],
    out_specs=pl.BlockSpec((NB, D), lambda i: (i, 0)),
    out_shape=jax.ShapeDtypeStruct((N, D), jnp.float32),
)


@jax.jit
def kernel(x, h, g, edge_idx, edge_type, W1, b1, weights_att):
    del x
    row = edge_idx[0]
    col = edge_idx[1]
    et = edge_type
    w1t = W1.T
    wa = w1t[:IN_H]
    wb = w1t[IN_H:2 * IN_H]
    wc = w1t[2 * IN_H:]
    watt = weights_att.reshape(HEADS, OUT)
    z1 = jnp.zeros((NP,), jnp.float32)
    z2 = jnp.zeros((NP, D), jnp.float32)

    u, v, at, gt, ag = _k1(h, g, wa, wb, wc, b1.reshape(1, D), watt)
    ev0, ev1, spart = _k2(row, col, et, at.reshape(-1), ag.reshape(-1), z1)
    accp, sap = _k4(row, col, et, ev0, ev1, spart, u, gt, z1, z2)
    acc0 = accp[0].reshape(NP, D)[:N]
    acc1 = accp[1].reshape(NP, D)[:N]
    sat = sap.reshape(NC * HEADS, NP).T[:N]
    out = _k5(acc0, acc1, sat, v)
    return out.reshape(N, HEADS, OUT)
